# baseline (device time: 16686 ns/iter reference)
import jax
import jax.numpy as jnp
from jax import lax
from jax.experimental import pallas as pl
from jax.experimental.pallas import tpu as pltpu

N_DEV = 8
N_TOK = 512
D = 256
H = 512
N_EXP = 32
E_LOCAL = N_EXP // N_DEV
ROWS = N_TOK // N_DEV


def kernel(x, router_W, route_idx, expert_W, shared_W):
    def body(
        x_ref,
        rw_ref,
        idx_ref,
        ew_ref,
        sw_ref,
        out_ref,
        xs_ref,
        wk_ref,
        send_buf,
        recv_buf,
        send_sems,
        recv_sems,
    ):
        my = lax.axis_index("i")

        barrier = pltpu.get_barrier_semaphore()
        for dd in range(1, N_DEV):
            pl.semaphore_signal(
                barrier,
                inc=1,
                device_id=((my + dd) % N_DEV,),
                device_id_type=pl.DeviceIdType.MESH,
            )

        xv = x_ref[...]
        scores = jnp.dot(xv, rw_ref[...], preferred_element_type=jnp.float32)
        smax = jnp.max(scores, axis=1, keepdims=True)
        ex = jnp.exp(scores - smax)
        probs = ex / jnp.sum(ex, axis=1, keepdims=True)

        idx = idx_ref[...]
        onehot = idx == lax.broadcasted_iota(jnp.int32, (N_TOK, N_EXP), 1)
        p_sel = jnp.sum(
            probs * onehot.astype(jnp.float32), axis=1, keepdims=True
        )

        for k in range(E_LOCAL):
            ck = p_sel * (idx == my * E_LOCAL + k).astype(jnp.float32)
            xs_ref[:, k * D : (k + 1) * D] = (xv * ck).astype(jnp.bfloat16)
            wk_ref[k * D : (k + 1) * D, :] = ew_ref[k].astype(jnp.bfloat16)

        pl.semaphore_wait(barrier, N_DEV - 1)

        rdmas = {}
        for dd in (2, 6, 3, 5, 1, 7, 4):
            dst = (my + dd) % N_DEV
            send_buf[dd] = jnp.dot(
                xs_ref[pl.ds(dst * ROWS, ROWS), :],
                wk_ref[...],
                preferred_element_type=jnp.float32,
            ).astype(jnp.bfloat16)
            rdma = pltpu.make_async_remote_copy(
                src_ref=send_buf.at[dd],
                dst_ref=recv_buf.at[dd],
                send_sem=send_sems.at[dd],
                recv_sem=recv_sems.at[dd],
                device_id=(dst,),
                device_id_type=pl.DeviceIdType.MESH,
            )
            rdma.start()
            rdmas[dd] = rdma

        acc = jnp.dot(
            xs_ref[pl.ds(my * ROWS, ROWS), :],
            wk_ref[...],
            preferred_element_type=jnp.float32,
        )
        acc = acc + jnp.dot(
            x_ref[pl.ds(my * ROWS, ROWS), :].astype(jnp.bfloat16),
            sw_ref[...].astype(jnp.bfloat16),
            preferred_element_type=jnp.float32,
        )

        for dd in range(1, N_DEV):
            rdmas[dd].wait_recv()
            acc = acc + recv_buf[dd].astype(jnp.float32)

        out_ref[...] = acc

        for dd in range(1, N_DEV):
            rdmas[dd].wait_send()

    return pl.pallas_call(
        body,
        out_shape=jax.ShapeDtypeStruct((ROWS, H), jnp.float32),
        in_specs=[pl.BlockSpec(memory_space=pltpu.VMEM)] * 5,
        out_specs=pl.BlockSpec(memory_space=pltpu.VMEM),
        scratch_shapes=[
            pltpu.VMEM((N_TOK, E_LOCAL * D), jnp.bfloat16),
            pltpu.VMEM((E_LOCAL * D, H), jnp.bfloat16),
            pltpu.VMEM((N_DEV, ROWS, H), jnp.bfloat16),
            pltpu.VMEM((N_DEV, ROWS, H), jnp.bfloat16),
            pltpu.SemaphoreType.DMA((N_DEV,)),
            pltpu.SemaphoreType.DMA((N_DEV,)),
        ],
        compiler_params=pltpu.CompilerParams(collective_id=0),
    )(x, router_W, route_idx, expert_W, shared_W)


# device time: 15959 ns/iter; 1.0456x vs baseline; 1.0456x over previous
import jax
import jax.numpy as jnp
from jax import lax
from jax.experimental import pallas as pl
from jax.experimental.pallas import tpu as pltpu

N_DEV = 8
N_TOK = 512
D = 256
H = 512
N_EXP = 32
E_LOCAL = N_EXP // N_DEV
ROWS = N_TOK // N_DEV


def kernel(x, router_W, route_idx, expert_W, shared_W):
    def body(
        x_ref,
        rw_ref,
        idx_ref,
        ew_ref,
        sw_ref,
        out_ref,
        xs_ref,
        wk_ref,
        send_buf,
        recv_buf,
        send_sems,
        recv_sems,
    ):
        my = lax.axis_index("i")

        barrier = pltpu.get_barrier_semaphore()
        for dd in range(1, N_DEV):
            pl.semaphore_signal(
                barrier,
                inc=1,
                device_id=((my + dd) % N_DEV,),
                device_id_type=pl.DeviceIdType.MESH,
            )

        xv = x_ref[...]
        scores = jnp.dot(xv, rw_ref[...], preferred_element_type=jnp.float32)
        smax = jnp.max(scores, axis=1, keepdims=True)
        ex = jnp.exp(scores - smax)
        probs = ex / jnp.sum(ex, axis=1, keepdims=True)

        idx = idx_ref[...]
        onehot = idx == lax.broadcasted_iota(jnp.int32, (N_TOK, N_EXP), 1)
        p_sel = jnp.sum(
            probs * onehot.astype(jnp.float32), axis=1, keepdims=True
        )

        for k in range(E_LOCAL):
            ck = p_sel * (idx == my * E_LOCAL + k).astype(jnp.float32)
            xs_ref[:, k * D : (k + 1) * D] = (xv * ck).astype(jnp.bfloat16)
            wk_ref[k * D : (k + 1) * D, :] = ew_ref[k].astype(jnp.bfloat16)

        pl.semaphore_wait(barrier, N_DEV - 1)

        rdmas = {}
        for dd in range(1, N_DEV):
            dst = (my + dd) % N_DEV
            send_buf[dd] = jnp.dot(
                xs_ref[pl.ds(dst * ROWS, ROWS), :],
                wk_ref[...],
                preferred_element_type=jnp.float32,
            ).astype(jnp.bfloat16)
            rdma = pltpu.make_async_remote_copy(
                src_ref=send_buf.at[dd],
                dst_ref=recv_buf.at[dd],
                send_sem=send_sems.at[dd],
                recv_sem=recv_sems.at[dd],
                device_id=(dst,),
                device_id_type=pl.DeviceIdType.MESH,
            )
            rdma.start()
            rdmas[dd] = rdma

        acc = jnp.dot(
            xs_ref[pl.ds(my * ROWS, ROWS), :],
            wk_ref[...],
            preferred_element_type=jnp.float32,
        )
        acc = acc + jnp.dot(
            x_ref[pl.ds(my * ROWS, ROWS), :].astype(jnp.bfloat16),
            sw_ref[...].astype(jnp.bfloat16),
            preferred_element_type=jnp.float32,
        )

        for dd in range(1, N_DEV):
            rdmas[dd].wait_recv()
            acc = acc + recv_buf[dd].astype(jnp.float32)

        out_ref[...] = acc

        for dd in range(1, N_DEV):
            rdmas[dd].wait_send()

    return pl.pallas_call(
        body,
        out_shape=jax.ShapeDtypeStruct((ROWS, H), jnp.float32),
        in_specs=[pl.BlockSpec(memory_space=pltpu.VMEM)] * 5,
        out_specs=pl.BlockSpec(memory_space=pltpu.VMEM),
        scratch_shapes=[
            pltpu.VMEM((N_TOK, E_LOCAL * D), jnp.bfloat16),
            pltpu.VMEM((E_LOCAL * D, H), jnp.bfloat16),
            pltpu.VMEM((N_DEV, ROWS, H), jnp.bfloat16),
            pltpu.VMEM((N_DEV, ROWS, H), jnp.bfloat16),
            pltpu.SemaphoreType.DMA((N_DEV,)),
            pltpu.SemaphoreType.DMA((N_DEV,)),
        ],
        compiler_params=pltpu.CompilerParams(collective_id=0),
    )(x, router_W, route_idx, expert_W, shared_W)
